# baseline (speedup 1.0000x reference)
import functools

import jax
import jax.numpy as jnp
from jax import lax
from jax.experimental import pallas as pl
from jax.experimental.pallas import tpu as pltpu
from jax.experimental.pallas import tpu_sc as plsc

E = 64
B = 16384
L = 12
TOK = B * L
NC, NS = 2, 16
NW = NC * NS
PER_W = TOK // NW
C = 128
CP = C // 2
N_CHUNKS = PER_W // C

BLK = 2048
N_BLKS = (TOK // 2) // BLK


BW = B // NW
CB = 64
N_CHUNKS_W = 6 * (BW // CB)


def _sc_gather_sum(ts2, ti2, ta2, tact2, tab_s, tab_i, tab_a, tab_act):
    mesh = plsc.VectorSubcoreMesh(core_axis_name="c", subcore_axis_name="s")

    buf_types = [pltpu.VMEM((CB, E), jnp.float32) for _ in range(16)]

    @functools.partial(
        pl.kernel,
        mesh=mesh,
        out_type=jax.ShapeDtypeStruct((TOK // 2, 2 * E), jnp.float32),
        compiler_params=pltpu.CompilerParams(use_tc_tiling_on_sc=False),
        scratch_types=[
            pltpu.VMEM((L, BW), jnp.int32),
            pltpu.VMEM((L, BW), jnp.int32),
            pltpu.VMEM((L, BW), jnp.int32),
            pltpu.VMEM((L, BW), jnp.int32),
        ] + buf_types + [
            pltpu.VMEM((CB, 2 * E), jnp.float32),
            pltpu.VMEM((CB, 2 * E), jnp.float32),
            pltpu.SemaphoreType.DMA,
            pltpu.SemaphoreType.DMA,
            pltpu.SemaphoreType.DMA,
            pltpu.SemaphoreType.DMA,
            pltpu.SemaphoreType.DMA,
        ],
    )
    def k(ts_h, ti_h, ta_h, tact_h, tabs_h, tabi_h, taba_h, tabact_h, out_h,
          s0, s1, s2, s3,
          b000, b001, b010, b011, b020, b021, b030, b031,
          b100, b101, b110, b111, b120, b121, b130, b131,
          o0, o1,
          sg0, sg1, so0, so1, ssl):
        wid = lax.axis_index("s") * NC + lax.axis_index("c")
        wb0 = wid * BW

        slabs = (s0, s1, s2, s3)
        toks = (ts_h, ti_h, ta_h, tact_h)
        tabs = (tabs_h, tabi_h, taba_h, tabact_h)
        bufs = (((b000, b001), (b010, b011), (b020, b021), (b030, b031)),
                ((b100, b101), (b110, b111), (b120, b121), (b130, b131)))
        obufs = (o0, o1)
        gsems = (sg0, sg1)
        osems = (so0, so1)

        for t in range(4):
            for l in range(L):
                pltpu.async_copy(toks[t].at[l, pl.ds(wb0, BW)],
                                 slabs[t].at[l], ssl)
        for t in range(4):
            for l in range(L):
                pltpu.make_async_copy(toks[t].at[l, pl.ds(0, BW)],
                                      slabs[t].at[l], ssl).wait()

        def fire(g, slot):
            jj = lax.div(g, 8)
            boff = lax.rem(g, 8) * CB
            for t in range(4):
                for h in range(2):
                    idx = slabs[t].at[2 * jj + h, pl.ds(boff, CB)]
                    pltpu.async_copy(tabs[t].at[idx], bufs[slot][t][h],
                                     gsems[slot])

        def wait_gathers(slot):
            for t in range(4):
                for h in range(2):
                    pltpu.make_async_copy(tabs[t].at[pl.ds(0, CB)],
                                          bufs[slot][t][h],
                                          gsems[slot]).wait()

        def wait_out(slot):
            pltpu.make_async_copy(obufs[slot],
                                  out_h.at[pl.ds(0, CB)], osems[slot]).wait()

        def do_sum(g, slot):
            jj = lax.div(g, 8)
            boff = lax.rem(g, 8) * CB
            bt = bufs[slot]
            ob = obufs[slot]

            def group(kk, carry):
                mv = [[jnp.where(
                    slabs[t][2 * jj + h, pl.ds(boff + kk * 16, 16)] == 0,
                    0.0, 1.0) for h in range(2)] for t in range(4)]
                for j in range(16):
                    b = 16 * kk + j
                    for h in range(2):
                        for q in range(4):
                            sl = pl.ds(q * 16, 16)
                            ob[b, pl.ds(h * E + q * 16, 16)] = (
                                mv[0][h][j] * bt[0][h][b, sl]
                                + mv[1][h][j] * bt[1][h][b, sl]
                                + mv[2][h][j] * bt[2][h][b, sl]
                                + mv[3][h][j] * bt[3][h][b, sl])
                return carry

            lax.fori_loop(0, CB // 16, group, 0)

        fire(0, 0)

        def chunk_pair(gg, carry):
            for slot in range(2):
                g = 2 * gg + slot

                @pl.when(g + 1 < N_CHUNKS_W)
                def _():
                    fire(g + 1, 1 - slot)

                wait_gathers(slot)

                @pl.when(g >= 2)
                def _():
                    wait_out(slot)

                do_sum(g, slot)
                jj = lax.div(g, 8)
                boff = lax.rem(g, 8) * CB
                prow = jj * B + wb0 + boff
                pltpu.async_copy(obufs[slot], out_h.at[pl.ds(prow, CB)],
                                 osems[slot])
            return carry

        lax.fori_loop(0, N_CHUNKS_W // 2, chunk_pair, 0)
        wait_out(0)
        wait_out(1)

    return k(ts2, ti2, ta2, tact2, tab_s, tab_i, tab_a, tab_act)


NBB = 512
N_TCB = B // NBB


def _tc_body(x0, x1, x2, x3, x4, x5, w_ref, o_ref):
    wv = w_ref[...]
    for j, xr in enumerate((x0, x1, x2, x3, x4, x5)):
        xj = xr[...]
        for h in range(2):
            xl = xj[:, h * E:(h + 1) * E]
            y = lax.dot_general(wv, xl, (((0,), (1,)), ((), ())))
            o_ref[2 * j + h] = jnp.maximum(y, 0.0)


def _tc_project(x, w):
    xspecs = [
        pl.BlockSpec((NBB, 2 * E), lambda i, j=j: (j * N_TCB + i, 0))
        for j in range(6)
    ]
    return pl.pallas_call(
        _tc_body,
        grid=(N_TCB,),
        in_specs=xspecs + [pl.BlockSpec((E, E), lambda i: (0, 0))],
        out_specs=pl.BlockSpec((L, E, NBB), lambda i: (0, 0, i)),
        out_shape=jax.ShapeDtypeStruct((L, E, B), jnp.float32),
    )(x, x, x, x, x, x, w)


def kernel(species_tokens, items_tokens, abilities_tokens, actions_tokens,
           species_table, items_table, abilities_table, actions_table,
           W_combine):
    tokst = [jnp.pad(t.astype(jnp.int32).T, ((0, 4), (0, 0))) for t in
             (species_tokens, items_tokens, abilities_tokens, actions_tokens)]

    combined = _sc_gather_sum(*tokst, species_table, items_table,
                              abilities_table, actions_table)

    out_t = _tc_project(combined, W_combine)
    return jnp.transpose(out_t, (2, 0, 1))


# speedup vs baseline: 1.4259x; 1.4259x over previous
import functools

import jax
import jax.numpy as jnp
from jax import lax
from jax.experimental import pallas as pl
from jax.experimental.pallas import tpu as pltpu
from jax.experimental.pallas import tpu_sc as plsc

E = 64
B = 16384
L = 12
TOK = B * L
NC, NS = 2, 16
NW = NC * NS
PER_W = TOK // NW
C = 128
CP = C // 2
N_CHUNKS = PER_W // C

BLK = 2048
N_BLKS = (TOK // 2) // BLK


def _sc_gather_sum(ts2, ti2, ta2, tact2, tab_s, tab_i, tab_a, tab_act):
    mesh = plsc.VectorSubcoreMesh(core_axis_name="c", subcore_axis_name="s")

    @functools.partial(
        pl.kernel,
        mesh=mesh,
        out_type=jax.ShapeDtypeStruct((TOK // 2, 2 * E), jnp.float32),
        compiler_params=pltpu.CompilerParams(use_tc_tiling_on_sc=False),
        scratch_types=[
            pltpu.VMEM((N_CHUNKS, C), jnp.int32),
            pltpu.VMEM((N_CHUNKS, C), jnp.int32),
            pltpu.VMEM((N_CHUNKS, C), jnp.int32),
            pltpu.VMEM((N_CHUNKS, C), jnp.int32),
            pltpu.VMEM((C, E), jnp.float32),
            pltpu.VMEM((C, E), jnp.float32),
            pltpu.VMEM((C, E), jnp.float32),
            pltpu.VMEM((C, E), jnp.float32),
            pltpu.VMEM((C, E), jnp.float32),
            pltpu.VMEM((C, E), jnp.float32),
            pltpu.VMEM((C, E), jnp.float32),
            pltpu.VMEM((C, E), jnp.float32),
            pltpu.VMEM((CP, 2 * E), jnp.float32),
            pltpu.VMEM((CP, 2 * E), jnp.float32),
            pltpu.VMEM((CP,), jnp.int32),
            pltpu.VMEM((CP,), jnp.int32),
            pltpu.SemaphoreType.DMA,
            pltpu.SemaphoreType.DMA,
            pltpu.SemaphoreType.DMA,
            pltpu.SemaphoreType.DMA,
        ],
    )
    def k(ts_h, ti_h, ta_h, tact_h, tabs_h, tabi_h, taba_h, tabact_h, out_h,
          s0, s1, s2, s3,
          b00, b01, b02, b03, b10, b11, b12, b13,
          o0, o1, io0, io1,
          sg0, sg1, so0, so1):
        wid = lax.axis_index("s") * NC + lax.axis_index("c")
        base_chunk = wid * N_CHUNKS

        slabs = (s0, s1, s2, s3)
        tabs = (tabs_h, tabi_h, taba_h, tabact_h)
        bufs = ((b00, b01, b02, b03), (b10, b11, b12, b13))
        obufs = (o0, o1)
        iobufs = (io0, io1)
        gsems = (sg0, sg1)
        osems = (so0, so1)

        pltpu.sync_copy(ts_h.at[pl.ds(base_chunk, N_CHUNKS)], s0)
        pltpu.sync_copy(ti_h.at[pl.ds(base_chunk, N_CHUNKS)], s1)
        pltpu.sync_copy(ta_h.at[pl.ds(base_chunk, N_CHUNKS)], s2)
        pltpu.sync_copy(tact_h.at[pl.ds(base_chunk, N_CHUNKS)], s3)

        def fire(g, slot):
            for t in range(4):
                pltpu.async_copy(tabs[t].at[slabs[t].at[g]],
                                 bufs[slot][t], gsems[slot])

        def wait_gathers(slot):
            for t in range(4):
                pltpu.make_async_copy(tabs[t].at[pl.ds(0, C)],
                                      bufs[slot][t], gsems[slot]).wait()

        def wait_out(slot):
            pltpu.make_async_copy(obufs[slot],
                                  out_h.at[pl.ds(0, CP)], osems[slot]).wait()

        def do_sum(g, slot):
            bt = bufs[slot]
            ob = obufs[slot]

            def group(k, carry):
                mv = [jnp.where(slabs[t][g, pl.ds(k * 16, 16)] == 0, 0.0, 1.0)
                      for t in range(4)]
                for j in range(16):
                    r = 16 * k + j
                    p = 8 * k + (j // 2)
                    off = (j % 2) * E
                    for q in range(4):
                        sl = pl.ds(q * 16, 16)
                        ob[p, pl.ds(off + q * 16, 16)] = (
                            mv[0][j] * bt[0][r, sl] + mv[1][j] * bt[1][r, sl]
                            + mv[2][j] * bt[2][r, sl]
                            + mv[3][j] * bt[3][r, sl])
                return carry

            lax.fori_loop(0, C // 16, group, 0)

        fire(0, 0)

        def chunk_pair(gg, carry):
            for slot in range(2):
                g = 2 * gg + slot

                @pl.when(g + 1 < N_CHUNKS)
                def _():
                    fire(g + 1, 1 - slot)

                wait_gathers(slot)

                @pl.when(g >= 2)
                def _():
                    wait_out(slot)

                do_sum(g, slot)
                p0 = (wid * N_CHUNKS + g) * CP
                for kk in range(CP // 16):
                    qv = p0 + kk * 16 + lax.iota(jnp.int32, 16)
                    iobufs[slot][pl.ds(kk * 16, 16)] = (
                        lax.rem(qv, 6) * B + lax.div(qv, 6))
                pltpu.async_copy(obufs[slot], out_h.at[iobufs[slot]],
                                 osems[slot])
            return carry

        lax.fori_loop(0, N_CHUNKS // 2, chunk_pair, 0)
        wait_out(0)
        wait_out(1)

    return k(ts2, ti2, ta2, tact2, tab_s, tab_i, tab_a, tab_act)


NBB = 512
N_TCB = B // NBB


def _tc_body(x0, x1, x2, x3, x4, x5, w_ref, o_ref):
    wv = w_ref[...]
    for j, xr in enumerate((x0, x1, x2, x3, x4, x5)):
        xj = xr[...]
        for h in range(2):
            xl = xj[:, h * E:(h + 1) * E]
            y = lax.dot_general(wv, xl, (((0,), (1,)), ((), ())))
            o_ref[2 * j + h] = jnp.maximum(y, 0.0)


def _tc_project(x, w):
    xspecs = [
        pl.BlockSpec((NBB, 2 * E), lambda i, j=j: (j * N_TCB + i, 0))
        for j in range(6)
    ]
    return pl.pallas_call(
        _tc_body,
        grid=(N_TCB,),
        in_specs=xspecs + [pl.BlockSpec((E, E), lambda i: (0, 0))],
        out_specs=pl.BlockSpec((L, E, NBB), lambda i: (0, 0, i)),
        out_shape=jax.ShapeDtypeStruct((L, E, B), jnp.float32),
    )(x, x, x, x, x, x, w)


def kernel(species_tokens, items_tokens, abilities_tokens, actions_tokens,
           species_table, items_table, abilities_table, actions_table,
           W_combine):
    toks = [t.reshape(-1).astype(jnp.int32) for t in
            (species_tokens, items_tokens, abilities_tokens, actions_tokens)]
    toks2d = [t.reshape(NW * N_CHUNKS, C) for t in toks]

    combined = _sc_gather_sum(*toks2d, species_table, items_table,
                              abilities_table, actions_table)

    out_t = _tc_project(combined, W_combine)
    return jnp.transpose(out_t, (2, 0, 1))


# speedup vs baseline: 1.5859x; 1.1122x over previous
import functools

import jax
import jax.numpy as jnp
from jax import lax
from jax.experimental import pallas as pl
from jax.experimental.pallas import tpu as pltpu
from jax.experimental.pallas import tpu_sc as plsc

E = 64
B = 16384
L = 12
TOK = B * L
NC, NS = 2, 16
NW = NC * NS
PER_W = TOK // NW
C = 128
CP = C // 2
N_CHUNKS = PER_W // C

BLK = 2048
N_BLKS = (TOK // 2) // BLK


def _sc_gather_sum(ts2, ti2, ta2, tact2, tab_s, tab_i, tab_a, tab_act):
    mesh = plsc.VectorSubcoreMesh(core_axis_name="c", subcore_axis_name="s")

    @functools.partial(
        pl.kernel,
        mesh=mesh,
        out_type=jax.ShapeDtypeStruct((TOK // 2, 2 * E), jnp.float32),
        compiler_params=pltpu.CompilerParams(use_tc_tiling_on_sc=False),
        scratch_types=[
            pltpu.VMEM((L, B // NW), jnp.int32),
            pltpu.VMEM((L, B // NW), jnp.int32),
            pltpu.VMEM((L, B // NW), jnp.int32),
            pltpu.VMEM((L, B // NW), jnp.int32),
            pltpu.VMEM((C, E), jnp.float32),
            pltpu.VMEM((C, E), jnp.float32),
            pltpu.VMEM((C, E), jnp.float32),
            pltpu.VMEM((C, E), jnp.float32),
            pltpu.VMEM((C, E), jnp.float32),
            pltpu.VMEM((C, E), jnp.float32),
            pltpu.VMEM((C, E), jnp.float32),
            pltpu.VMEM((C, E), jnp.float32),
            pltpu.VMEM((C, E), jnp.float32),
            pltpu.VMEM((C, E), jnp.float32),
            pltpu.SemaphoreType.DMA,
            pltpu.SemaphoreType.DMA,
            pltpu.SemaphoreType.DMA,
            pltpu.SemaphoreType.DMA,
        ],
    )
    def k(ts_h, ti_h, ta_h, tact_h, tabs_h, tabi_h, taba_h, tabact_h, out_h,
          s0, s1, s2, s3,
          b00, b01, b02, b03, b10, b11, b12, b13,
          o0, o1,
          sg0, sg1, so0, so1):
        wid = lax.axis_index("s") * NC + lax.axis_index("c")
        wb0 = wid * (B // NW)

        slabs = (s0, s1, s2, s3)
        toks = (ts_h, ti_h, ta_h, tact_h)
        tabs = (tabs_h, tabi_h, taba_h, tabact_h)
        bufs = ((b00, b01, b02, b03), (b10, b11, b12, b13))
        obufs = (o0, o1)
        gsems = (sg0, sg1)
        osems = (so0, so1)

        for t in range(4):
            for l in range(L):
                pltpu.async_copy(toks[t].at[l, pl.ds(wb0, B // NW)],
                                 slabs[t].at[l], sg0)
        for t in range(4):
            for l in range(L):
                pltpu.make_async_copy(toks[t].at[l, pl.ds(0, B // NW)],
                                      slabs[t].at[l], sg0).wait()

        def fire(g, slot):
            l = lax.div(g, 4)
            boff = lax.rem(g, 4) * C
            for t in range(4):
                idx = slabs[t].at[l, pl.ds(boff, C)]
                pltpu.async_copy(tabs[t].at[idx], bufs[slot][t],
                                 gsems[slot])

        def wait_gathers(slot):
            for t in range(4):
                pltpu.make_async_copy(tabs[t].at[pl.ds(0, C)],
                                      bufs[slot][t], gsems[slot]).wait()

        def wait_out(slot):
            pltpu.make_async_copy(
                obufs[slot], out_h.at[pl.ds(0, C), pl.ds(0, E)],
                osems[slot]).wait()

        def do_sum(g, slot):
            l = lax.div(g, 4)
            boff = lax.rem(g, 4) * C
            bt = bufs[slot]
            ob = obufs[slot]

            def group(kk, carry):
                mv = [jnp.where(
                    slabs[t][l, pl.ds(boff + kk * 16, 16)] == 0, 0.0, 1.0)
                    for t in range(4)]
                for j in range(16):
                    r = 16 * kk + j
                    for q in range(4):
                        sl = pl.ds(q * 16, 16)
                        ob[r, sl] = (
                            mv[0][j] * bt[0][r, sl] + mv[1][j] * bt[1][r, sl]
                            + mv[2][j] * bt[2][r, sl]
                            + mv[3][j] * bt[3][r, sl])
                return carry

            lax.fori_loop(0, C // 16, group, 0)

        fire(0, 0)

        def chunk_pair(gg, carry):
            for slot in range(2):
                g = 2 * gg + slot

                @pl.when(g + 1 < N_CHUNKS)
                def _():
                    fire(g + 1, 1 - slot)

                wait_gathers(slot)

                @pl.when(g >= 2)
                def _():
                    wait_out(slot)

                do_sum(g, slot)
                l = lax.div(g, 4)
                boff = lax.rem(g, 4) * C
                prow = lax.div(l, 2) * B + wb0 + boff
                hoff = lax.rem(l, 2) * E
                pltpu.async_copy(
                    obufs[slot],
                    out_h.at[pl.ds(prow, C), pl.ds(hoff, E)],
                    osems[slot])
            return carry

        lax.fori_loop(0, N_CHUNKS // 2, chunk_pair, 0)
        wait_out(0)
        wait_out(1)

    return k(ts2, ti2, ta2, tact2, tab_s, tab_i, tab_a, tab_act)


NBB = 512
N_TCB = B // NBB


def _tc_body(x0, x1, x2, x3, x4, x5, w_ref, o_ref):
    wv = w_ref[...]
    for j, xr in enumerate((x0, x1, x2, x3, x4, x5)):
        xj = xr[...]
        for h in range(2):
            xl = xj[:, h * E:(h + 1) * E]
            y = lax.dot_general(wv, xl, (((0,), (1,)), ((), ())))
            o_ref[2 * j + h] = jnp.maximum(y, 0.0)


def _tc_project(x, w):
    xspecs = [
        pl.BlockSpec((NBB, 2 * E), lambda i, j=j: (j * N_TCB + i, 0))
        for j in range(6)
    ]
    return pl.pallas_call(
        _tc_body,
        grid=(N_TCB,),
        in_specs=xspecs + [pl.BlockSpec((E, E), lambda i: (0, 0))],
        out_specs=pl.BlockSpec((L, E, NBB), lambda i: (0, 0, i)),
        out_shape=jax.ShapeDtypeStruct((L, E, B), jnp.float32),
    )(x, x, x, x, x, x, w)


def kernel(species_tokens, items_tokens, abilities_tokens, actions_tokens,
           species_table, items_table, abilities_table, actions_table,
           W_combine):
    tokst = [jnp.pad(t.astype(jnp.int32).T, ((0, 4), (0, 0))) for t in
             (species_tokens, items_tokens, abilities_tokens, actions_tokens)]

    combined = _sc_gather_sum(*tokst, species_table, items_table,
                              abilities_table, actions_table)

    out_t = _tc_project(combined, W_combine)
    return jnp.transpose(out_t, (2, 0, 1))


# speedup vs baseline: 1.5877x; 1.0011x over previous
import functools

import jax
import jax.numpy as jnp
from jax import lax
from jax.experimental import pallas as pl
from jax.experimental.pallas import tpu as pltpu
from jax.experimental.pallas import tpu_sc as plsc

E = 64
B = 16384
L = 12
TOK = B * L
NC, NS = 2, 16
NW = NC * NS
PER_W = TOK // NW
C = 128
CP = C // 2
N_CHUNKS = PER_W // C

BLK = 2048
N_BLKS = (TOK // 2) // BLK


def _sc_gather_sum(ts2, ti2, ta2, tact2, tab_s, tab_i, tab_a, tab_act):
    mesh = plsc.VectorSubcoreMesh(core_axis_name="c", subcore_axis_name="s")

    @functools.partial(
        pl.kernel,
        mesh=mesh,
        out_type=jax.ShapeDtypeStruct((TOK // 2, 2 * E), jnp.float32),
        compiler_params=pltpu.CompilerParams(use_tc_tiling_on_sc=False),
        scratch_types=[
            pltpu.VMEM((L, B // NW), jnp.int32),
            pltpu.VMEM((L, B // NW), jnp.int32),
            pltpu.VMEM((L, B // NW), jnp.int32),
            pltpu.VMEM((L, B // NW), jnp.int32),
            pltpu.VMEM((C, E), jnp.float32),
            pltpu.VMEM((C, E), jnp.float32),
            pltpu.VMEM((C, E), jnp.float32),
            pltpu.VMEM((C, E), jnp.float32),
            pltpu.VMEM((C, E), jnp.float32),
            pltpu.VMEM((C, E), jnp.float32),
            pltpu.VMEM((C, E), jnp.float32),
            pltpu.VMEM((C, E), jnp.float32),
            pltpu.VMEM((C, E), jnp.float32),
            pltpu.VMEM((C, E), jnp.float32),
            pltpu.SemaphoreType.DMA,
            pltpu.SemaphoreType.DMA,
            pltpu.SemaphoreType.DMA,
            pltpu.SemaphoreType.DMA,
        ],
    )
    def k(ts_h, ti_h, ta_h, tact_h, tabs_h, tabi_h, taba_h, tabact_h, out_h,
          s0, s1, s2, s3,
          b00, b01, b02, b03, b10, b11, b12, b13,
          o0, o1,
          sg0, sg1, so0, so1):
        wid = lax.axis_index("s") * NC + lax.axis_index("c")
        wb0 = wid * (B // NW)

        slabs = (s0, s1, s2, s3)
        toks = (ts_h, ti_h, ta_h, tact_h)
        tabs = (tabs_h, tabi_h, taba_h, tabact_h)
        bufs = ((b00, b01, b02, b03), (b10, b11, b12, b13))
        obufs = (o0, o1)
        gsems = (sg0, sg1)
        osems = (so0, so1)

        for t in range(4):
            for l in range(L):
                pltpu.async_copy(toks[t].at[l, pl.ds(wb0, B // NW)],
                                 slabs[t].at[l], sg0)
        for t in range(4):
            for l in range(L):
                pltpu.make_async_copy(toks[t].at[l, pl.ds(0, B // NW)],
                                      slabs[t].at[l], sg0).wait()

        def fire(g, slot):
            l = lax.div(g, 4)
            boff = lax.rem(g, 4) * C
            for t in range(4):
                idx = slabs[t].at[l, pl.ds(boff, C)]
                pltpu.async_copy(tabs[t].at[idx], bufs[slot][t],
                                 gsems[slot])

        def wait_gathers(slot):
            for t in range(4):
                pltpu.make_async_copy(tabs[t].at[pl.ds(0, C)],
                                      bufs[slot][t], gsems[slot]).wait()

        def wait_out(slot):
            pltpu.make_async_copy(
                obufs[slot], out_h.at[pl.ds(0, C), pl.ds(0, E)],
                osems[slot]).wait()

        def do_sum(g, slot):
            l = lax.div(g, 4)
            boff = lax.rem(g, 4) * C
            bt = bufs[slot]
            ob = obufs[slot]

            def group(kk, carry):
                mv = [jnp.where(
                    slabs[t][l, pl.ds(boff + kk * 16, 16)] == 0, 0.0, 1.0)
                    for t in range(4)]
                for j in range(16):
                    r = 16 * kk + j
                    for q in range(4):
                        sl = pl.ds(q * 16, 16)
                        ob[r, sl] = (
                            mv[0][j] * bt[0][r, sl] + mv[1][j] * bt[1][r, sl]
                            + mv[2][j] * bt[2][r, sl]
                            + mv[3][j] * bt[3][r, sl])
                return carry

            lax.fori_loop(0, C // 16, group, 0)

        fire(0, 0)

        def chunk_pair(gg, carry):
            for slot in range(2):
                g = 2 * gg + slot

                @pl.when(g + 1 < N_CHUNKS)
                def _():
                    fire(g + 1, 1 - slot)

                wait_gathers(slot)

                @pl.when(g >= 2)
                def _():
                    wait_out(slot)

                do_sum(g, slot)
                l = lax.div(g, 4)
                boff = lax.rem(g, 4) * C
                prow = lax.div(l, 2) * B + wb0 + boff
                hoff = lax.rem(l, 2) * E
                pltpu.async_copy(
                    obufs[slot],
                    out_h.at[pl.ds(prow, C), pl.ds(hoff, E)],
                    osems[slot])
            return carry

        lax.fori_loop(0, N_CHUNKS // 2, chunk_pair, 0)
        wait_out(0)
        wait_out(1)

    return k(ts2, ti2, ta2, tact2, tab_s, tab_i, tab_a, tab_act)


NBB = 512
N_TCB = B // NBB


def _tc_body(x0, x1, x2, x3, x4, x5, w_ref, o_ref):
    wv = w_ref[...]
    for j, xr in enumerate((x0, x1, x2, x3, x4, x5)):
        xj = xr[...]
        for h in range(2):
            xl = xj[:, h * E:(h + 1) * E]
            y = lax.dot_general(wv, xl, (((0,), (1,)), ((), ())))
            o_ref[2 * j + h] = jnp.maximum(y, 0.0)


def _tc_project(x, w):
    xspecs = [
        pl.BlockSpec((NBB, 2 * E), lambda i, j=j: (j * N_TCB + i, 0))
        for j in range(6)
    ]
    return pl.pallas_call(
        _tc_body,
        grid=(N_TCB,),
        in_specs=xspecs + [pl.BlockSpec((E, E), lambda i: (0, 0))],
        out_specs=pl.BlockSpec((L, E, NBB), lambda i: (0, 0, i)),
        out_shape=jax.ShapeDtypeStruct((L, E, B), jnp.float32),
    )(x, x, x, x, x, x, w)


def kernel(species_tokens, items_tokens, abilities_tokens, actions_tokens,
           species_table, items_table, abilities_table, actions_table,
           W_combine):
    tokst = [jnp.pad(jnp.maximum(t.astype(jnp.int32).T, 0), ((0, 4), (0, 0)))
             for t in
             (species_tokens, items_tokens, abilities_tokens, actions_tokens)]

    combined = _sc_gather_sum(*tokst, species_table, items_table,
                              abilities_table, actions_table)

    out_t = _tc_project(combined, W_combine)
    return jnp.transpose(out_t, (2, 0, 1))
